# baseline (speedup 1.0000x reference)
import functools

import jax
import jax.numpy as jnp
import numpy as np
from jax import lax
from jax.experimental import pallas as pl
from jax.experimental.pallas import tpu as pltpu
from jax.experimental.pallas import tpu_sc as plsc


def _slow_indices(T: int) -> list[int]:
    return [int(v) for v in np.linspace(0.0, T - 1, T // 4).astype(np.int32)]


@functools.lru_cache(maxsize=None)
def _make_gather(C: int, T: int, D: int):
    idx = _slow_indices(T)
    rows = [c * T + i for c in range(C) for i in idx]
    n_rows = len(rows)

    info = plsc.get_sparse_core_info()
    n_workers = info.num_cores * info.num_subcores

    mesh = plsc.VectorSubcoreMesh(core_axis_name="c", subcore_axis_name="s")

    @functools.partial(
        pl.kernel,
        out_type=jax.ShapeDtypeStruct((n_rows, D), jnp.float32),
        mesh=mesh,
    )
    def gather_rows(src_hbm, out_hbm):
        wid = lax.axis_index("s") * info.num_cores + lax.axis_index("c")
        for i in range(n_rows):
            @pl.when(wid == i % n_workers)
            def _copy(i=i):
                pltpu.sync_copy(src_hbm.at[rows[i]], out_hbm.at[i])

    return gather_rows


def kernel(frames):
    C, T, H, W = frames.shape
    D = H * W
    flat = frames.reshape(C * T, D)
    slow = _make_gather(C, T, D)(flat).reshape(C, T // 4, H, W)
    return (slow, frames)


# speedup vs baseline: 2.8592x; 2.8592x over previous
import functools

import jax
import jax.numpy as jnp
import numpy as np
from jax import lax
from jax.experimental import pallas as pl
from jax.experimental.pallas import tpu as pltpu
from jax.experimental.pallas import tpu_sc as plsc


def _slow_indices(T: int) -> list[int]:
    return [int(v) for v in np.linspace(0.0, T - 1, T // 4).astype(np.int32)]


@functools.lru_cache(maxsize=None)
def _make_gather(C: int, T: int, D: int):
    idx = _slow_indices(T)
    rows = [c * T + i for c in range(C) for i in idx]
    n_rows = len(rows)

    info = plsc.get_sparse_core_info()
    n_workers = info.num_cores * info.num_subcores

    chunks_per_row = 4
    assert D % chunks_per_row == 0
    ch = D // chunks_per_row
    assert ch % 8 == 0
    n_chunks = n_rows * chunks_per_row
    assert n_chunks % n_workers == 0
    per_w = n_chunks // n_workers

    mesh = plsc.VectorSubcoreMesh(core_axis_name="c", subcore_axis_name="s")

    @functools.partial(
        pl.kernel,
        out_type=jax.ShapeDtypeStruct((n_rows * D,), jnp.float32),
        mesh=mesh,
        scratch_types=(
            [pltpu.VMEM((per_w * ch,), jnp.float32)]
            + [pltpu.SemaphoreType.DMA] * (2 * per_w)
        ),
    )
    def gather_rows(src_hbm, out_hbm, buf, *sems):
        gsem, ssem = sems[:per_w], sems[per_w:]
        wid = lax.axis_index("s") * info.num_cores + lax.axis_index("c")
        for w in range(n_workers):
            @pl.when(wid == w)
            def _copy(w=w):
                handles = []
                for j in range(per_w):
                    k = per_w * w + j
                    src_off = rows[k // chunks_per_row] * D + (k % chunks_per_row) * ch
                    handles.append(pltpu.async_copy(
                        src_hbm.at[pl.ds(src_off, ch)], buf.at[pl.ds(j * ch, ch)], gsem[j]))
                out_handles = []
                for j in range(per_w):
                    handles[j].wait()
                    k = per_w * w + j
                    out_handles.append(pltpu.async_copy(
                        buf.at[pl.ds(j * ch, ch)], out_hbm.at[pl.ds(k * ch, ch)], ssem[j]))
                for h in out_handles:
                    h.wait()

    return gather_rows


def kernel(frames):
    C, T, H, W = frames.shape
    D = H * W
    flat = frames.reshape(C * T * D)
    slow = _make_gather(C, T, D)(flat).reshape(C, T // 4, H, W)
    return (slow, frames)


# speedup vs baseline: 5.2574x; 1.8388x over previous
import functools

import jax
import jax.numpy as jnp
import numpy as np
from jax import lax
from jax.experimental import pallas as pl
from jax.experimental.pallas import tpu as pltpu
from jax.experimental.pallas import tpu_sc as plsc


@functools.lru_cache(maxsize=None)
def _make_gather(C: int, T: int, H: int, W: int):
    n_slow = T // 4
    lin = [int(v) for v in np.linspace(0.0, T - 1, n_slow).astype(np.int32)]
    assert lin == [(t * (T - 1)) // (n_slow - 1) for t in range(n_slow)]

    info = plsc.get_sparse_core_info()
    n_workers = info.num_cores * info.num_subcores

    n_chunks = C * n_slow * 4
    assert n_chunks % n_workers == 0
    per_w = n_chunks // n_workers
    hq = H // 4
    assert hq % 8 == 0

    mesh = plsc.VectorSubcoreMesh(core_axis_name="c", subcore_axis_name="s")

    @functools.partial(
        pl.kernel,
        out_type=jax.ShapeDtypeStruct((C, n_slow, H, W), jnp.float32),
        mesh=mesh,
        scratch_types=(
            [pltpu.VMEM((per_w * hq, W), jnp.float32)]
            + [pltpu.SemaphoreType.DMA] * (2 * per_w)
        ),
    )
    def gather_frames(src_hbm, out_hbm, buf, *sems):
        gsem, ssem = sems[:per_w], sems[per_w:]
        wid = lax.axis_index("s") * info.num_cores + lax.axis_index("c")

        def chunk_coords(j):
            k = per_w * wid + j
            r = k // 4
            q = k % 4
            cc = r // n_slow
            ts = r % n_slow
            st = (ts * (T - 1)) // (n_slow - 1)
            return cc, ts, st, q

        handles = []
        for j in range(per_w):
            cc, ts, st, q = chunk_coords(j)
            handles.append(pltpu.async_copy(
                src_hbm.at[cc, st, pl.ds(q * hq, hq), :],
                buf.at[pl.ds(j * hq, hq), :],
                gsem[j]))
        out_handles = []
        for j in range(per_w):
            handles[j].wait()
            cc, ts, st, q = chunk_coords(j)
            out_handles.append(pltpu.async_copy(
                buf.at[pl.ds(j * hq, hq), :],
                out_hbm.at[cc, ts, pl.ds(q * hq, hq), :],
                ssem[j]))
        for h in out_handles:
            h.wait()

    return gather_frames


def kernel(frames):
    C, T, H, W = frames.shape
    slow = _make_gather(C, T, H, W)(frames)
    return (slow, frames)


# speedup vs baseline: 5.7295x; 1.0898x over previous
import functools

import jax
import jax.numpy as jnp
import numpy as np
from jax import lax
from jax.experimental import pallas as pl
from jax.experimental.pallas import tpu as pltpu
from jax.experimental.pallas import tpu_sc as plsc


@functools.lru_cache(maxsize=None)
def _make_gather(C: int, T: int, H: int, W: int):
    n_slow = T // 4
    lin = [int(v) for v in np.linspace(0.0, T - 1, n_slow).astype(np.int32)]
    assert lin == [(t * (T - 1)) // (n_slow - 1) for t in range(n_slow)]

    info = plsc.get_sparse_core_info()
    n_workers = info.num_cores * info.num_subcores

    n_chunks = C * n_slow * 4
    assert n_chunks % n_workers == 0
    per_w = n_chunks // n_workers
    hq = H // 4
    assert hq % 8 == 0

    mesh = plsc.VectorSubcoreMesh(core_axis_name="c", subcore_axis_name="s")

    @functools.partial(
        pl.kernel,
        out_type=jax.ShapeDtypeStruct((C, n_slow, H, W), jnp.float32),
        mesh=mesh,
        scratch_types=(
            [pltpu.VMEM((per_w * hq, W), jnp.float32)]
            + [pltpu.SemaphoreType.DMA] * (2 * per_w)
        ),
    )
    def gather_frames(src_hbm, out_hbm, buf, *sems):
        gsem, ssem = sems[:per_w], sems[per_w:]
        wid = lax.axis_index("s") * info.num_cores + lax.axis_index("c")

        def chunk_coords(j):
            k = per_w * wid + j
            r = k // 4
            q = k % 4
            cc = r // n_slow
            ts = r % n_slow
            st = (ts * (T - 1)) // (n_slow - 1)
            return cc, ts, st, q

        handles = []
        for j in range(per_w):
            cc, ts, st, q = chunk_coords(j)
            handles.append(pltpu.async_copy(
                src_hbm.at[cc, st, pl.ds(q * hq, hq), :],
                buf.at[pl.ds(j * hq, hq), :],
                gsem[j]))
        out_handles = []
        for j in range(per_w):
            handles[j].wait()
            cc, ts, st, q = chunk_coords(j)
            out_handles.append(pltpu.async_copy(
                buf.at[pl.ds(j * hq, hq), :],
                out_hbm.at[cc, ts, pl.ds(q * hq, hq), :],
                ssem[j]))
        for h in out_handles:
            h.wait()

    return gather_frames


@functools.lru_cache(maxsize=None)
def _make_fast_copy(C: int, T: int, H: int, W: int):
    tb = 8
    assert T % tb == 0

    def body(src_ref, out_ref):
        out_ref[...] = src_ref[...]

    return pl.pallas_call(
        body,
        grid=(C, T // tb),
        in_specs=[pl.BlockSpec((1, tb, H, W), lambda c, t: (c, t, 0, 0))],
        out_specs=pl.BlockSpec((1, tb, H, W), lambda c, t: (c, t, 0, 0)),
        out_shape=jax.ShapeDtypeStruct((C, T, H, W), jnp.float32),
    )


def kernel(frames):
    C, T, H, W = frames.shape
    slow = _make_gather(C, T, H, W)(frames)
    fast = _make_fast_copy(C, T, H, W)(frames)
    return (slow, fast)


# speedup vs baseline: 6.1742x; 1.0776x over previous
import functools

import jax
import jax.numpy as jnp
import numpy as np
from jax import lax
from jax.experimental import pallas as pl
from jax.experimental.pallas import tpu as pltpu
from jax.experimental.pallas import tpu_sc as plsc


@functools.lru_cache(maxsize=None)
def _make_gather(C: int, T: int, H: int, W: int):
    n_slow = T // 4
    lin = [int(v) for v in np.linspace(0.0, T - 1, n_slow).astype(np.int32)]
    assert lin == [(t * (T - 1)) // (n_slow - 1) for t in range(n_slow)]

    info = plsc.get_sparse_core_info()
    n_workers = info.num_cores * info.num_subcores

    n_sel = C * n_slow
    assert n_sel <= n_workers

    mesh = plsc.VectorSubcoreMesh(core_axis_name="c", subcore_axis_name="s")

    @functools.partial(
        pl.kernel,
        out_type=jax.ShapeDtypeStruct((C, n_slow, H, W), jnp.float32),
        mesh=mesh,
        scratch_types=[
            pltpu.VMEM((H, W), jnp.float32),
            pltpu.SemaphoreType.DMA,
            pltpu.SemaphoreType.DMA,
        ],
    )
    def gather_frames(src_hbm, out_hbm, buf, gsem, ssem):
        wid = lax.axis_index("s") * info.num_cores + lax.axis_index("c")

        @pl.when(wid < n_sel)
        def _copy():
            cc = wid // n_slow
            ts = wid % n_slow
            st = (ts * (T - 1)) // (n_slow - 1)
            pltpu.async_copy(src_hbm.at[cc, st], buf, gsem).wait()
            pltpu.async_copy(buf, out_hbm.at[cc, ts], ssem).wait()

    return gather_frames


@functools.lru_cache(maxsize=None)
def _make_fast_copy(C: int, T: int, H: int, W: int):
    tb = 16
    assert T % tb == 0

    def body(src_ref, out_ref):
        out_ref[...] = src_ref[...]

    return pl.pallas_call(
        body,
        grid=(C, T // tb),
        in_specs=[pl.BlockSpec((1, tb, H, W), lambda c, t: (c, t, 0, 0))],
        out_specs=pl.BlockSpec((1, tb, H, W), lambda c, t: (c, t, 0, 0)),
        out_shape=jax.ShapeDtypeStruct((C, T, H, W), jnp.float32),
    )


def kernel(frames):
    C, T, H, W = frames.shape
    slow = _make_gather(C, T, H, W)(frames)
    fast = _make_fast_copy(C, T, H, W)(frames)
    return (slow, fast)


# speedup vs baseline: 6.3504x; 1.0285x over previous
import functools

import jax
import jax.numpy as jnp
import numpy as np
from jax import lax
from jax.experimental import pallas as pl
from jax.experimental.pallas import tpu as pltpu
from jax.experimental.pallas import tpu_sc as plsc


@functools.lru_cache(maxsize=None)
def _make_gather(C: int, T: int, H: int, W: int):
    n_slow = T // 4
    lin = [int(v) for v in np.linspace(0.0, T - 1, n_slow).astype(np.int32)]
    assert lin == [(t * (T - 1)) // (n_slow - 1) for t in range(n_slow)]

    info = plsc.get_sparse_core_info()
    n_workers = info.num_subcores

    n_chunks = C * n_slow * 4
    assert n_chunks % n_workers == 0
    per_w = n_chunks // n_workers
    hq = H // 4
    assert hq % 8 == 0

    mesh = plsc.VectorSubcoreMesh(
        core_axis_name="c", subcore_axis_name="s", num_cores=1)

    @functools.partial(
        pl.kernel,
        out_type=jax.ShapeDtypeStruct((C, n_slow, H, W), jnp.float32),
        mesh=mesh,
        scratch_types=(
            [pltpu.VMEM((per_w * hq, W), jnp.float32)]
            + [pltpu.SemaphoreType.DMA] * (2 * per_w)
        ),
    )
    def gather_frames(src_hbm, out_hbm, buf, *sems):
        gsem, ssem = sems[:per_w], sems[per_w:]
        wid = lax.axis_index("s")

        def chunk_coords(j):
            k = per_w * wid + j
            r = k // 4
            q = k % 4
            cc = r // n_slow
            ts = r % n_slow
            st = (ts * (T - 1)) // (n_slow - 1)
            return cc, ts, st, q

        handles = []
        for j in range(per_w):
            cc, ts, st, q = chunk_coords(j)
            handles.append(pltpu.async_copy(
                src_hbm.at[cc, st, pl.ds(q * hq, hq), :],
                buf.at[pl.ds(j * hq, hq), :],
                gsem[j]))
        out_handles = []
        for j in range(per_w):
            handles[j].wait()
            cc, ts, st, q = chunk_coords(j)
            out_handles.append(pltpu.async_copy(
                buf.at[pl.ds(j * hq, hq), :],
                out_hbm.at[cc, ts, pl.ds(q * hq, hq), :],
                ssem[j]))
        for h in out_handles:
            h.wait()

    return gather_frames


@functools.lru_cache(maxsize=None)
def _make_fast_copy(C: int, T: int, H: int, W: int):
    tb = 16
    assert T % tb == 0

    def body(src_ref, out_ref):
        out_ref[...] = src_ref[...]

    return pl.pallas_call(
        body,
        grid=(C, T // tb),
        in_specs=[pl.BlockSpec((1, tb, H, W), lambda c, t: (c, t, 0, 0))],
        out_specs=pl.BlockSpec((1, tb, H, W), lambda c, t: (c, t, 0, 0)),
        out_shape=jax.ShapeDtypeStruct((C, T, H, W), jnp.float32),
    )


def kernel(frames):
    C, T, H, W = frames.shape
    slow = _make_gather(C, T, H, W)(frames)
    fast = _make_fast_copy(C, T, H, W)(frames)
    return (slow, fast)
